# baseline (speedup 1.0000x reference)
import jax
import jax.numpy as jnp
from jax.experimental import pallas as pl
from jax.experimental.pallas import tpu as pltpu


def _gc_kernel(adj_ref, x_ref, w_ref, out_ref):
    xw = jnp.dot(x_ref[...], w_ref[...], preferred_element_type=jnp.float32)
    out_ref[...] = jnp.dot(adj_ref[...], xw,
                           preferred_element_type=jnp.float32).astype(out_ref.dtype)


def kernel(adjacency, nodes, weights):
    B, W, N, _ = adjacency.shape
    Fin = nodes.shape[-1]
    Fout = weights.shape[-1]
    itemsize = jnp.dtype(adjacency.dtype).itemsize

    flops = 2 * B * W * (N * N * Fout + N * Fin * Fout)
    bytes_accessed = itemsize * (adjacency.size + nodes.size + weights.size
                                 + B * W * N * Fout)
    cost = pl.CostEstimate(flops=flops, transcendentals=0,
                           bytes_accessed=bytes_accessed)

    return pl.pallas_call(
        _gc_kernel,
        out_shape=jax.ShapeDtypeStruct((B, W, N, Fout), nodes.dtype),
        grid_spec=pl.GridSpec(
            grid=(B, W),
            in_specs=[
                pl.BlockSpec((pl.Squeezed(), pl.Squeezed(), N, N),
                             lambda b, w: (b, w, 0, 0)),
                pl.BlockSpec((pl.Squeezed(), pl.Squeezed(), N, Fin),
                             lambda b, w: (b, w, 0, 0)),
                pl.BlockSpec((pl.Squeezed(), Fin, Fout),
                             lambda b, w: (w, 0, 0)),
            ],
            out_specs=pl.BlockSpec((pl.Squeezed(), pl.Squeezed(), N, Fout),
                                   lambda b, w: (b, w, 0, 0)),
        ),
        compiler_params=pltpu.CompilerParams(
            dimension_semantics=("parallel", "parallel"),
        ),
        cost_estimate=cost,
    )(adjacency, nodes, weights)


# speedup vs baseline: 1.3763x; 1.3763x over previous
import jax
import jax.numpy as jnp
from jax.experimental import pallas as pl
from jax.experimental.pallas import tpu as pltpu


def _gc_kernel(adj_ref, x_ref, w_ref, out_ref):
    W = w_ref.shape[0]
    for w in range(W):
        xw = jnp.einsum("bnf,fo->bno", x_ref[:, w], w_ref[w],
                        preferred_element_type=jnp.float32)
        y = jnp.einsum("bij,bjo->bio", adj_ref[:, w], xw,
                       preferred_element_type=jnp.float32)
        out_ref[:, w] = y.astype(out_ref.dtype)


def kernel(adjacency, nodes, weights):
    B, W, N, _ = adjacency.shape
    Fin = nodes.shape[-1]
    Fout = weights.shape[-1]
    itemsize = jnp.dtype(adjacency.dtype).itemsize

    flops = 2 * B * W * (N * N * Fout + N * Fin * Fout)
    bytes_accessed = itemsize * (adjacency.size + nodes.size + weights.size
                                 + B * W * N * Fout)
    cost = pl.CostEstimate(flops=flops, transcendentals=0,
                           bytes_accessed=bytes_accessed)

    BB = B // 2
    return pl.pallas_call(
        _gc_kernel,
        out_shape=jax.ShapeDtypeStruct((B, W, N, Fout), nodes.dtype),
        grid_spec=pl.GridSpec(
            grid=(2,),
            in_specs=[
                pl.BlockSpec((BB, W, N, N), lambda i: (i, 0, 0, 0)),
                pl.BlockSpec((BB, W, N, Fin), lambda i: (i, 0, 0, 0)),
                pl.BlockSpec((W, Fin, Fout), lambda i: (0, 0, 0)),
            ],
            out_specs=pl.BlockSpec((BB, W, N, Fout), lambda i: (i, 0, 0, 0)),
        ),
        compiler_params=pltpu.CompilerParams(
            dimension_semantics=("parallel",),
            vmem_limit_bytes=60 * 1024 * 1024,
        ),
        cost_estimate=cost,
    )(adjacency, nodes, weights)
